# baseline (speedup 1.0000x reference)
import functools

import jax
import jax.numpy as jnp
from jax import lax
from jax.experimental import pallas as pl
from jax.experimental.pallas import tpu as pltpu
from jax.experimental.pallas import tpu_sc as plsc

_NC = 2
_NS = 16
_NW = _NC * _NS
_B = 128
_WIN = 16
_FRAC0 = 0.5


def _seg_body(chunks0, chunks1, rows_per_tile,
              x_hbm, src_hbm, dst_hbm, zeros_hbm, out_hbm,
              srcw_a, srcw_b, dstw_a, dstw_b, rows_a, rows_b, acc,
              sem_a, sem_b, sem_sa, sem_sb, sem_wa, sem_wb):
    cid = lax.axis_index("c")
    sid = lax.axis_index("s")
    wid = cid * _NS + sid
    nch = jnp.where(cid == 0, chunks0, chunks1)
    half = pl.multiple_of(nch // 2, _WIN)
    nwin = half // _WIN

    def gxa(idx_ref):
        pltpu.async_copy(x_hbm.at[idx_ref], rows_a, sem_a)

    def gxb(idx_ref):
        pltpu.async_copy(x_hbm.at[idx_ref], rows_b, sem_b)

    def wait(buf, sem):
        pltpu.make_async_copy(x_hbm.at[pl.ds(0, _B)], buf, sem).wait()

    def wwait(w4, sem):
        pltpu.make_async_copy(dst_hbm.at[wid, pl.ds(0, _WIN)],
                              w4.at[0], sem).wait()

    pltpu.sync_copy(zeros_hbm.at[pl.ds(sid * rows_per_tile, rows_per_tile)],
                    acc.at[pl.ds(sid * rows_per_tile, rows_per_tile)])
    pltpu.async_copy(src_hbm.at[wid, pl.ds(0, _WIN)], srcw_a.at[0], sem_sa)
    pltpu.async_copy(src_hbm.at[wid, pl.ds(half, _WIN)], srcw_b.at[0], sem_sb)
    pltpu.async_copy(dst_hbm.at[wid, pl.ds(0, _WIN)], dstw_a.at[0], sem_wa)
    pltpu.async_copy(dst_hbm.at[wid, pl.ds(half, _WIN)], dstw_b.at[0], sem_wb)
    plsc.subcore_barrier()

    wwait(srcw_a, sem_sa)
    wwait(srcw_b, sem_sb)
    gxa(srcw_a.at[0, 0])
    gxb(srcw_b.at[0, 0])

    def win_body(w, carry):
        p = lax.rem(w, 2)
        wwait(dstw_a, sem_wa)
        wwait(dstw_b, sem_wb)
        wn = jnp.minimum(w + 1, nwin - 1)
        basen = pl.multiple_of(wn * _WIN, _WIN)
        pltpu.async_copy(src_hbm.at[wid, pl.ds(basen, _WIN)],
                         srcw_a.at[1 - p], sem_sa)
        pltpu.async_copy(src_hbm.at[wid, pl.ds(half + basen, _WIN)],
                         srcw_b.at[1 - p], sem_sb)
        pltpu.async_copy(dst_hbm.at[wid, pl.ds(basen, _WIN)],
                         dstw_a.at[1 - p], sem_wa)
        pltpu.async_copy(dst_hbm.at[wid, pl.ds(half + basen, _WIN)],
                         dstw_b.at[1 - p], sem_wb)

        def body(j, carry2):
            wait(rows_a, sem_a)
            pltpu.sync_copy(rows_a, acc.at[dstw_a.at[p, j]], add=True)

            @pl.when(j < _WIN - 1)
            def _():
                gxa(srcw_a.at[p, j + 1])

            @pl.when(j == _WIN - 1)
            def _():
                wwait(srcw_a, sem_sa)
                gxa(srcw_a.at[1 - p, 0])

            wait(rows_b, sem_b)
            pltpu.sync_copy(rows_b, acc.at[dstw_b.at[p, j]], add=True)

            @pl.when(j < _WIN - 1)
            def _():
                gxb(srcw_b.at[p, j + 1])

            @pl.when(j == _WIN - 1)
            def _():
                wwait(srcw_b, sem_sb)
                gxb(srcw_b.at[1 - p, 0])

            return carry2

        lax.fori_loop(0, _WIN, body, carry)
        return carry

    lax.fori_loop(0, nwin, win_body, 0)
    wait(rows_a, sem_a)
    wait(rows_b, sem_b)
    wwait(dstw_a, sem_wa)
    wwait(dstw_b, sem_wb)
    plsc.subcore_barrier()

    pltpu.sync_copy(acc.at[pl.ds(sid * rows_per_tile, rows_per_tile)],
                    out_hbm.at[cid, pl.ds(sid * rows_per_tile, rows_per_tile)])


def _sc_segment_sum(x, src3, dst3, zeros, chunks0, chunks1):
    _, d = x.shape
    nrows_pad = zeros.shape[0]
    rows_per_tile = nrows_pad // _NS
    mesh = plsc.VectorSubcoreMesh(core_axis_name="c", subcore_axis_name="s")
    f = pl.kernel(
        functools.partial(_seg_body, chunks0, chunks1, rows_per_tile),
        out_type=jax.ShapeDtypeStruct((_NC, nrows_pad, d), jnp.float32),
        mesh=mesh,
        scratch_types=[
            pltpu.VMEM((2, _WIN, _B), jnp.int32),
            pltpu.VMEM((2, _WIN, _B), jnp.int32),
            pltpu.VMEM((2, _WIN, _B), jnp.int32),
            pltpu.VMEM((2, _WIN, _B), jnp.int32),
            pltpu.VMEM((_B, d), jnp.float32),
            pltpu.VMEM((_B, d), jnp.float32),
            pltpu.VMEM_SHARED((nrows_pad, d), jnp.float32),
            pltpu.SemaphoreType.DMA,
            pltpu.SemaphoreType.DMA,
            pltpu.SemaphoreType.DMA,
            pltpu.SemaphoreType.DMA,
            pltpu.SemaphoreType.DMA,
            pltpu.SemaphoreType.DMA,
        ],
    )
    return f(x, src3, dst3, zeros)


def _z_body(p0_ref, p1_ref, x_ref, wrel_ref, b2_ref, wroot_ref, z_ref):
    agg = p0_ref[...] + p1_ref[...]
    z_ref[...] = (
        jnp.dot(agg, wrel_ref[...], preferred_element_type=jnp.float32)
        + jnp.dot(x_ref[...], wroot_ref[...], preferred_element_type=jnp.float32)
        + b2_ref[...]
    )


def _zcompute(p0, p1, x, W_rel, b2, W_root):
    n = x.shape[0]
    dh = W_rel.shape[1]
    return pl.pallas_call(
        _z_body,
        out_shape=jax.ShapeDtypeStruct((n, dh), jnp.float32),
    )(p0, p1, x, W_rel, b2, W_root)


def _dec_body(zr_ref, zf_ref, o_ref):
    logits = lax.dot_general(
        zr_ref[...], zf_ref[...], (((1,), (1,)), ((), ())),
        preferred_element_type=jnp.float32,
    )
    o_ref[...] = jax.nn.sigmoid(logits)


def _decoder(z, rows_blk):
    n, dh = z.shape
    grid = (n // rows_blk,)
    return pl.pallas_call(
        _dec_body,
        grid=grid,
        in_specs=[
            pl.BlockSpec((rows_blk, dh), lambda i: (i, 0)),
            pl.BlockSpec((n, dh), lambda i: (0, 0)),
        ],
        out_specs=pl.BlockSpec((rows_blk, n), lambda i: (i, 0)),
        out_shape=jax.ShapeDtypeStruct((n, n), jnp.float32),
    )(z, z)


def kernel(x, edge_index, W_rel, b_rel, W_root):
    n, d = x.shape
    dh = W_rel.shape[1]
    e = edge_index.shape[1]

    csum = -(-e // (_NS * _B * 2 * _WIN)) * (2 * _WIN)
    c0 = min(max(int(round(csum * _FRAC0 / (2 * _WIN))) * (2 * _WIN),
                 2 * _WIN), csum - 2 * _WIN)
    c1 = csum - c0
    chmax = max(c0, c1)
    e_pad = _NS * _B * csum
    src = edge_index[0]
    dst = edge_index[1]
    if e_pad != e:
        pad = e_pad - e
        src = jnp.concatenate([src, jnp.zeros((pad,), jnp.int32)])
        dst = jnp.concatenate([dst, jnp.full((pad,), n, jnp.int32)])

    def _percore(a):
        e0 = _NS * c0 * _B
        p0 = a[:e0].reshape(_NS, c0, _B)
        p1 = a[e0:].reshape(_NS, c1, _B)
        p0 = jnp.pad(p0, ((0, 0), (0, chmax - c0), (0, 0)))
        p1 = jnp.pad(p1, ((0, 0), (0, chmax - c1), (0, 0)))
        return jnp.concatenate([p0[None], p1[None]]).reshape(_NW, chmax, _B)

    src3 = _percore(src)
    dst3 = _percore(dst)
    nrows_pad = -(-(n + 1) // (_NS * 8)) * (_NS * 8)
    zeros = jnp.zeros((nrows_pad, d), jnp.float32)

    parts = _sc_segment_sum(x, src3, dst3, zeros, c0, c1)
    z = _zcompute(parts[0, :n], parts[1, :n], x, W_rel,
                  b_rel.reshape(1, dh), W_root)
    adj = _decoder(z, 400)
    return adj, z


# speedup vs baseline: 1.2354x; 1.2354x over previous
import functools

import jax
import jax.numpy as jnp
from jax import lax
from jax.experimental import pallas as pl
from jax.experimental.pallas import tpu as pltpu
from jax.experimental.pallas import tpu_sc as plsc

_NC = 2
_NS = 16
_NW = _NC * _NS
_B = 128


def _seg_body(chunks, rows_per_tile,
              x_hbm, src_hbm, dst_hbm, zeros_hbm, out_hbm,
              src_v, dst_v, rows_v, acc, sem):
    cid = lax.axis_index("c")
    sid = lax.axis_index("s")
    wid = cid * _NS + sid

    pltpu.sync_copy(zeros_hbm.at[pl.ds(sid * rows_per_tile, rows_per_tile)],
                    acc.at[pl.ds(sid * rows_per_tile, rows_per_tile)])
    pltpu.sync_copy(src_hbm.at[wid], src_v)
    pltpu.sync_copy(dst_hbm.at[wid], dst_v)
    plsc.subcore_barrier()

    def body(c, carry):
        pltpu.async_copy(x_hbm.at[src_v.at[c]], rows_v, sem).wait()
        pltpu.sync_copy(rows_v, acc.at[dst_v.at[c]], add=True)
        return carry

    lax.fori_loop(0, chunks, body, 0)
    plsc.subcore_barrier()

    pltpu.sync_copy(acc.at[pl.ds(sid * rows_per_tile, rows_per_tile)],
                    out_hbm.at[cid, pl.ds(sid * rows_per_tile, rows_per_tile)])


def _sc_segment_sum(x, src3, dst3, zeros):
    _, d = x.shape
    nrows_pad = zeros.shape[0]
    chunks = src3.shape[1]
    rows_per_tile = nrows_pad // _NS
    mesh = plsc.VectorSubcoreMesh(core_axis_name="c", subcore_axis_name="s")
    f = pl.kernel(
        functools.partial(_seg_body, chunks, rows_per_tile),
        out_type=jax.ShapeDtypeStruct((_NC, nrows_pad, d), jnp.float32),
        mesh=mesh,
        scratch_types=[
            pltpu.VMEM((chunks, _B), jnp.int32),
            pltpu.VMEM((chunks, _B), jnp.int32),
            pltpu.VMEM((_B, d), jnp.float32),
            pltpu.VMEM_SHARED((nrows_pad, d), jnp.float32),
            pltpu.SemaphoreType.DMA,
        ],
    )
    return f(x, src3, dst3, zeros)


def _z_body(parts_ref, x_ref, wrel_ref, b2_ref, wroot_ref, z_ref):
    agg = parts_ref[0] + parts_ref[1]
    z_ref[...] = (
        jnp.dot(agg, wrel_ref[...], preferred_element_type=jnp.float32)
        + jnp.dot(x_ref[...], wroot_ref[...], preferred_element_type=jnp.float32)
        + b2_ref[...]
    )


def _zcompute(parts, x, W_rel, b2, W_root):
    n, d = x.shape
    dh = W_rel.shape[1]
    return pl.pallas_call(
        _z_body,
        grid=(1,),
        in_specs=[
            pl.BlockSpec((2, n, d), lambda i: (0, 0, 0)),
            pl.BlockSpec((n, d), lambda i: (0, 0)),
            pl.BlockSpec(W_rel.shape, lambda i: (0, 0)),
            pl.BlockSpec(b2.shape, lambda i: (0, 0)),
            pl.BlockSpec(W_root.shape, lambda i: (0, 0)),
        ],
        out_specs=pl.BlockSpec((n, dh), lambda i: (0, 0)),
        out_shape=jax.ShapeDtypeStruct((n, dh), jnp.float32),
    )(parts, x, W_rel, b2, W_root)


def _dec_body(zr_ref, zf_ref, o_ref):
    logits = lax.dot_general(
        zr_ref[...], zf_ref[...], (((1,), (1,)), ((), ())),
        preferred_element_type=jnp.float32,
    )
    o_ref[...] = jax.nn.sigmoid(logits)


def _decoder(z, rows_blk):
    n, dh = z.shape
    grid = (n // rows_blk,)
    return pl.pallas_call(
        _dec_body,
        grid=grid,
        in_specs=[
            pl.BlockSpec((rows_blk, dh), lambda i: (i, 0)),
            pl.BlockSpec((n, dh), lambda i: (0, 0)),
        ],
        out_specs=pl.BlockSpec((rows_blk, n), lambda i: (i, 0)),
        out_shape=jax.ShapeDtypeStruct((n, n), jnp.float32),
    )(z, z)


def kernel(x, edge_index, W_rel, b_rel, W_root):
    n, d = x.shape
    dh = W_rel.shape[1]
    e = edge_index.shape[1]

    chunks = -(-e // (_NW * _B))
    e_pad = _NW * chunks * _B
    src = edge_index[0]
    dst = edge_index[1]
    if e_pad != e:
        pad = e_pad - e
        src = jnp.concatenate([src, jnp.zeros((pad,), jnp.int32)])
        dst = jnp.concatenate([dst, jnp.full((pad,), n, jnp.int32)])
    src3 = src.reshape(_NW, chunks, _B)
    dst3 = dst.reshape(_NW, chunks, _B)
    nrows_pad = -(-(n + 1) // (_NS * 8)) * (_NS * 8)
    zeros = jnp.zeros((nrows_pad, d), jnp.float32)

    parts = _sc_segment_sum(x, src3, dst3, zeros)
    z = _zcompute(parts, x, W_rel, b_rel.reshape(1, dh), W_root)
    adj = _decoder(z, 400)
    return adj, z


# speedup vs baseline: 1.5616x; 1.2640x over previous
import functools

import jax
import jax.numpy as jnp
from jax import lax
from jax.experimental import pallas as pl
from jax.experimental.pallas import tpu as pltpu
from jax.experimental.pallas import tpu_sc as plsc

_NC = 2
_NS = 16
_NW = _NC * _NS
_B = 128


def _seg_body(chunks, rows_per_tile,
              x_hbm, src_hbm, dst_hbm, zeros_hbm, out_hbm,
              src_v, dst_v, rows_v, acc, sem):
    cid = lax.axis_index("c")
    sid = lax.axis_index("s")
    wid = cid * _NS + sid

    pltpu.sync_copy(zeros_hbm.at[pl.ds(sid * rows_per_tile, rows_per_tile)],
                    acc.at[pl.ds(sid * rows_per_tile, rows_per_tile)])
    pltpu.sync_copy(src_hbm.at[wid], src_v)
    pltpu.sync_copy(dst_hbm.at[wid], dst_v)
    plsc.subcore_barrier()

    def body(c, carry):
        pltpu.async_copy(x_hbm.at[src_v.at[c]], rows_v, sem).wait()
        pltpu.sync_copy(rows_v, acc.at[dst_v.at[c]], add=True)
        return carry

    lax.fori_loop(0, chunks, body, 0)
    plsc.subcore_barrier()

    pltpu.sync_copy(acc.at[pl.ds(sid * rows_per_tile, rows_per_tile)],
                    out_hbm.at[cid, pl.ds(sid * rows_per_tile, rows_per_tile)])


def _sc_segment_sum(x, src3, dst3, zeros):
    _, d = x.shape
    nrows_pad = zeros.shape[0]
    chunks = src3.shape[1]
    rows_per_tile = nrows_pad // _NS
    mesh = plsc.VectorSubcoreMesh(core_axis_name="c", subcore_axis_name="s")
    f = pl.kernel(
        functools.partial(_seg_body, chunks, rows_per_tile),
        out_type=jax.ShapeDtypeStruct((_NC, nrows_pad, d), jnp.float32),
        mesh=mesh,
        compiler_params=pltpu.CompilerParams(use_tc_tiling_on_sc=False),
        scratch_types=[
            pltpu.VMEM((chunks, _B), jnp.int32),
            pltpu.VMEM((chunks, _B), jnp.int32),
            pltpu.VMEM((_B, d), jnp.float32),
            pltpu.VMEM_SHARED((nrows_pad, d), jnp.float32),
            pltpu.SemaphoreType.DMA,
        ],
    )
    return f(x, src3, dst3, zeros)


def _mm2_body(x_ref, wrel_ref, b2_ref, wroot_ref, y_ref, r_ref):
    xv = x_ref[...]
    y_ref[...] = jnp.dot(xv, wrel_ref[...], preferred_element_type=jnp.float32)
    r_ref[...] = (
        jnp.dot(xv, wroot_ref[...], preferred_element_type=jnp.float32)
        + b2_ref[...]
    )


def _mm2(x, W_rel, b2, W_root):
    n = x.shape[0]
    dh = W_rel.shape[1]
    return pl.pallas_call(
        _mm2_body,
        out_shape=(
            jax.ShapeDtypeStruct((n, dh), jnp.float32),
            jax.ShapeDtypeStruct((n, dh), jnp.float32),
        ),
    )(x, W_rel, b2, W_root)


def _z_body(parts_ref, r_ref, z_ref):
    z_ref[...] = parts_ref[0] + parts_ref[1] + r_ref[...]


def _zcompute(parts, r):
    n, dh = r.shape
    return pl.pallas_call(
        _z_body,
        grid=(1,),
        in_specs=[
            pl.BlockSpec((2, n, dh), lambda i: (0, 0, 0)),
            pl.BlockSpec((n, dh), lambda i: (0, 0)),
        ],
        out_specs=pl.BlockSpec((n, dh), lambda i: (0, 0)),
        out_shape=jax.ShapeDtypeStruct((n, dh), jnp.float32),
    )(parts, r)


def _dec_body(zr_ref, zf_ref, o_ref):
    logits = lax.dot_general(
        zr_ref[...], zf_ref[...], (((1,), (1,)), ((), ())),
        preferred_element_type=jnp.float32,
    )
    o_ref[...] = jax.nn.sigmoid(logits)


def _decoder(z, rows_blk):
    n, dh = z.shape
    grid = (n // rows_blk,)
    return pl.pallas_call(
        _dec_body,
        grid=grid,
        in_specs=[
            pl.BlockSpec((rows_blk, dh), lambda i: (i, 0)),
            pl.BlockSpec((n, dh), lambda i: (0, 0)),
        ],
        out_specs=pl.BlockSpec((rows_blk, n), lambda i: (i, 0)),
        out_shape=jax.ShapeDtypeStruct((n, n), jnp.float32),
    )(z, z)


def kernel(x, edge_index, W_rel, b_rel, W_root):
    n, d = x.shape
    dh = W_rel.shape[1]
    e = edge_index.shape[1]

    chunks = -(-e // (_NW * _B))
    e_pad = _NW * chunks * _B
    src = edge_index[0]
    dst = edge_index[1]
    if e_pad != e:
        pad = e_pad - e
        src = jnp.concatenate([src, jnp.zeros((pad,), jnp.int32)])
        dst = jnp.concatenate([dst, jnp.full((pad,), n, jnp.int32)])
    src3 = src.reshape(_NW, chunks, _B)
    dst3 = dst.reshape(_NW, chunks, _B)
    nrows_pad = -(-(n + 1) // (_NS * 8)) * (_NS * 8)
    zeros = jnp.zeros((nrows_pad, dh), jnp.float32)

    y, r = _mm2(x, W_rel, b_rel.reshape(1, dh), W_root)
    parts = _sc_segment_sum(y, src3, dst3, zeros)
    z = _zcompute(parts, r)
    adj = _decoder(z, 400)
    return adj, z
